# baseline (speedup 1.0000x reference)
import functools

import jax
import jax.numpy as jnp
from jax import lax
from jax.experimental import pallas as pl
from jax.experimental.pallas import tpu as pltpu
from jax.experimental.pallas import tpu_sc as plsc

_USER_NUM = 500000
_LANES = 16


def _make_sc_kernel(B, D, nw):
    b_per_w = B // nw
    n_groups = b_per_w // _LANES

    mesh = plsc.VectorSubcoreMesh(core_axis_name="c", subcore_axis_name="s")

    @functools.partial(
        pl.kernel,
        mesh=mesh,
        compiler_params=pltpu.CompilerParams(
            needs_layout_passes=False, use_tc_tiling_on_sc=False),
        out_type=jax.ShapeDtypeStruct((B,), jnp.float32),
        scratch_types=[
            pltpu.VMEM((b_per_w,), jnp.int32),
            pltpu.VMEM((b_per_w,), jnp.int32),
            pltpu.VMEM((D, _LANES * _LANES), jnp.float32),
            pltpu.VMEM((D, _LANES * _LANES), jnp.float32),
            pltpu.VMEM((b_per_w,), jnp.float32),
            pltpu.SemaphoreType.DMA,
        ],
    )
    def sc_kernel(emb_hbm, u_hbm, v_hbm, out_hbm, idx_u, idx_v, ublk,
                  vblk, out_v, sem):
        wid = lax.axis_index("s") * 2 + lax.axis_index("c")
        base = wid * b_per_w

        pltpu.sync_copy(u_hbm.at[pl.ds(base, b_per_w)], idx_u)
        pltpu.sync_copy(v_hbm.at[pl.ds(base, b_per_w)], idx_v)

        iota = lax.iota(jnp.int32, _LANES)

        def step(g, carry):
            sl = pl.ds(g * _LANES, _LANES)
            nu_vec = idx_u[sl]
            nv_vec = idx_v[sl]
            nu_base = nu_vec & ~15
            nv_base = nv_vec & ~15
            descs = []
            for lane in range(_LANES):
                dst = pl.ds(lane * _LANES, _LANES)
                ustart = pl.multiple_of(nu_base[lane], _LANES)
                descs.append(pltpu.async_copy(
                    emb_hbm.at[:, pl.ds(ustart, _LANES)],
                    ublk.at[:, dst], sem))
                vstart = pl.multiple_of(nv_base[lane], _LANES)
                descs.append(pltpu.async_copy(
                    emb_hbm.at[:, pl.ds(vstart, _LANES)],
                    vblk.at[:, dst], sem))
            for dsc in descs:
                dsc.wait()

            ulane = iota * _LANES + (nu_vec & 15)
            vlane = iota * _LANES + (nv_vec & 15)
            acc = jnp.zeros((_LANES,), jnp.float32)
            for d in range(D):
                dcol = jnp.full((_LANES,), d, jnp.int32)
                gu = plsc.load_gather(ublk, [dcol, ulane])
                gv = plsc.load_gather(vblk, [dcol, vlane])
                acc = acc + gu * gv
            out_v[sl] = acc
            return carry

        lax.fori_loop(0, n_groups, step, 0)

        pltpu.sync_copy(out_v, out_hbm.at[pl.ds(base, b_per_w)])

    return sc_kernel


def kernel(u, v, embedding):
    B = u.shape[0]
    D = embedding.shape[1]
    info = plsc.get_sparse_core_info()
    nw = info.num_cores * info.num_subcores

    emb_t = embedding.T
    u1 = u.astype(jnp.int32)
    v1 = v.astype(jnp.int32) + _USER_NUM

    sc = _make_sc_kernel(B, D, nw)
    return sc(emb_t, u1, v1)


# speedup vs baseline: 1.0364x; 1.0364x over previous
import functools

import jax
import jax.numpy as jnp
from jax import lax
from jax.experimental import pallas as pl
from jax.experimental.pallas import tpu as pltpu
from jax.experimental.pallas import tpu_sc as plsc

_USER_NUM = 500000
_LANES = 16


def _make_sc_kernel(B, D, nw):
    b_per_w = B // nw
    n_chunks = b_per_w // 128
    n_groups = b_per_w // _LANES

    mesh = plsc.VectorSubcoreMesh(core_axis_name="c", subcore_axis_name="s")

    @functools.partial(
        pl.kernel,
        mesh=mesh,
        compiler_params=pltpu.CompilerParams(
            needs_layout_passes=False, use_tc_tiling_on_sc=False),
        out_type=jax.ShapeDtypeStruct((B,), jnp.float32),
        scratch_types=[
            pltpu.VMEM((b_per_w,), jnp.int32),
            pltpu.VMEM((b_per_w,), jnp.int32),
            pltpu.VMEM((D, b_per_w), jnp.float32),
            pltpu.VMEM((D, b_per_w), jnp.float32),
            pltpu.VMEM((b_per_w,), jnp.float32),
            pltpu.SemaphoreType.DMA,
        ],
    )
    def sc_kernel(emb_hbm, u_hbm, v_hbm, out_hbm, idx_u, idx_v, u_cols,
                  v_cols, out_v, sem):
        wid = lax.axis_index("s") * 2 + lax.axis_index("c")
        base = wid * b_per_w

        pltpu.sync_copy(u_hbm.at[pl.ds(base, b_per_w)], idx_u)
        pltpu.sync_copy(v_hbm.at[pl.ds(base, b_per_w)], idx_v)

        def fire(d, carry):
            for c in range(n_chunks):
                csl = pl.ds(c * 128, 128)
                pltpu.async_copy(emb_hbm.at[d].at[idx_u.at[csl]],
                                 u_cols.at[d, csl], sem)
                pltpu.async_copy(emb_hbm.at[d].at[idx_v.at[csl]],
                                 v_cols.at[d, csl], sem)
            return carry

        lax.fori_loop(0, D, fire, 0)

        pltpu.make_async_copy(
            emb_hbm.at[:, pl.ds(0, b_per_w)], u_cols, sem).wait()
        pltpu.make_async_copy(
            emb_hbm.at[:, pl.ds(0, b_per_w)], v_cols, sem).wait()

        def group(g, carry):
            sl = pl.ds(g * _LANES, _LANES)
            acc = u_cols[0, sl] * v_cols[0, sl]
            for d in range(1, D):
                acc = acc + u_cols[d, sl] * v_cols[d, sl]
            out_v[sl] = acc
            return carry

        lax.fori_loop(0, n_groups, group, 0)

        pltpu.sync_copy(out_v, out_hbm.at[pl.ds(base, b_per_w)])

    return sc_kernel


def kernel(u, v, embedding):
    B = u.shape[0]
    D = embedding.shape[1]
    info = plsc.get_sparse_core_info()
    nw = info.num_cores * info.num_subcores

    emb_t = embedding.T
    u1 = u.astype(jnp.int32)
    v1 = v.astype(jnp.int32) + _USER_NUM

    sc = _make_sc_kernel(B, D, nw)
    return sc(emb_t, u1, v1)
